# baseline (speedup 1.0000x reference)
import functools

import jax
import jax.numpy as jnp
from jax import lax
from jax.experimental import pallas as pl
from jax.experimental.pallas import tpu as pltpu
from jax.experimental.pallas import tpu_sc as plsc

_LABEL_SMOOTHING = 0.1
_CONFIDENCE = 1.0 - _LABEL_SMOOTHING
_RB = 32
_NC = 2
_NSUB = 16
_NW = _NC * _NSUB
_B_SC = 512
_CHUNK = 2048


def _xlogy(x):
    safe = jnp.where(x > 0, x, 1.0)
    return jnp.where(x > 0, x * jnp.log(safe), 0.0)


def _xlogy_const(x):
    import math
    return x * math.log(x) if x > 0 else 0.0


def _dense_body(h_ref, out_ref, res_ref, *, b):
    j = pl.program_id(0)
    blk = out_ref[...]
    h = h_ref[...]
    colsum = jnp.sum(blk, axis=0, keepdims=True)
    res_ref[...] = jnp.full((1, 1, 128), -jnp.sum(colsum * h),
                            dtype=jnp.float32)

    @pl.when(j == 0)
    def _():
        res_ref[...] += jnp.full((1, 1, 128), b * jnp.sum(_xlogy(h)),
                                 dtype=jnp.float32)


def _dense_partial(one_hot, output, rows):
    b, v = output.shape
    nb = rows // _RB
    return pl.pallas_call(
        functools.partial(_dense_body, b=b),
        grid=(nb,),
        in_specs=[
            pl.BlockSpec((1, v), lambda j: (0, 0)),
            pl.BlockSpec((_RB, v), lambda j: (j, 0)),
        ],
        out_specs=pl.BlockSpec((1, 1, 128), lambda j: (j, 0, 0)),
        out_shape=jax.ShapeDtypeStruct((nb, 1, 128), jnp.float32),
        compiler_params=pltpu.CompilerParams(
            dimension_semantics=("arbitrary",),
        ),
    )(one_hot, output)


def _sc_rowsum(output):
    b, v = output.shape
    rows_pw = (b - _B_SC) // _NW
    trs_pw = rows_pw // 8
    n_full = (v // 128) // (_CHUNK // 128)
    v_pad = -(-v // 128) * 128
    tail_dma = v_pad - n_full * _CHUNK
    tail_valid = v - n_full * _CHUNK
    nch = trs_pw * n_full

    mesh = plsc.VectorSubcoreMesh(core_axis_name="c", subcore_axis_name="s")

    @functools.partial(
        pl.kernel,
        out_type=jax.ShapeDtypeStruct((_NW, 16), jnp.float32),
        mesh=mesh,
        scratch_types=[
            pltpu.VMEM((8, _CHUNK), jnp.float32),
            pltpu.VMEM((8, _CHUNK), jnp.float32),
            pltpu.VMEM((8, tail_dma), jnp.float32),
            pltpu.VMEM((16,), jnp.float32),
            pltpu.SemaphoreType.DMA,
            pltpu.SemaphoreType.DMA,
            pltpu.SemaphoreType.DMA,
        ],
    )
    def rowsum_kernel(out_hbm, o_hbm, buf0, buf1, tbuf, acc_ref,
                      sem0, sem1, sem2):
        w = lax.axis_index("s") * _NC + lax.axis_index("c")
        row_base = _B_SC + w * rows_pw
        acc_ref[...] = jnp.zeros((16,), jnp.float32)

        def chunk_src(g):
            row = pl.multiple_of(row_base + (g // n_full) * 8, 8)
            col = pl.multiple_of((g % n_full) * _CHUNK, 128)
            return out_hbm.at[pl.ds(row, 8), pl.ds(col, _CHUNK)]

        def accum(bref, ncols):
            @pl.loop(0, ncols // 32)
            def _(cc):
                col = cc * 32
                s = None
                for dc in (0, 16):
                    vals = [bref[r, pl.ds(col + dc, 16)] for r in range(8)]
                    t = ((vals[0] + vals[1]) + (vals[2] + vals[3])) + (
                        (vals[4] + vals[5]) + (vals[6] + vals[7]))
                    s = t if s is None else s + t
                acc_ref[...] += s

        pltpu.async_copy(chunk_src(0), buf0, sem0)
        pltpu.async_copy(chunk_src(1), buf1, sem1)

        @pl.loop(0, nch // 2)
        def _(kk):
            g = kk * 2
            pltpu.make_async_copy(chunk_src(g), buf0, sem0).wait()
            accum(buf0, _CHUNK)

            @pl.when(g + 2 < nch)
            def _():
                pltpu.async_copy(chunk_src(g + 2), buf0, sem0)

            pltpu.make_async_copy(chunk_src(g + 1), buf1, sem1).wait()
            accum(buf1, _CHUNK)

            @pl.when(g + 3 < nch)
            def _():
                pltpu.async_copy(chunk_src(g + 3), buf1, sem1)

        for tr in range(trs_pw):
            row = pl.multiple_of(row_base + tr * 8, 8)
            col = pl.multiple_of(n_full * _CHUNK, 128)
            pltpu.async_copy(
                out_hbm.at[pl.ds(row, 8), pl.ds(col, tail_dma)],
                tbuf, sem2,
            ).wait()
            accum(tbuf, tail_valid)

        pltpu.sync_copy(acc_ref, o_hbm.at[w])

    return rowsum_kernel(output)


def _sc_gather(output, cols128):
    b, v = output.shape
    per_core = b // _NC

    mesh = plsc.ScalarSubcoreMesh(axis_name="c", num_cores=_NC)

    @functools.partial(
        pl.kernel,
        out_type=jax.ShapeDtypeStruct((8 * b, 128), jnp.float32),
        mesh=mesh,
        scratch_types=[
            pltpu.SMEM((per_core,), jnp.int32),
            pltpu.SemaphoreType.DMA,
            pltpu.SemaphoreType.DMA,
        ],
    )
    def gather_kernel(out_hbm, c128_hbm, g_hbm, tbuf, sem_t, sem_g):
        cid = lax.axis_index("c")
        base = cid * per_core
        pltpu.async_copy(
            c128_hbm.at[pl.ds(base, per_core)], tbuf, sem_t
        ).wait()

        @pl.loop(0, per_core)
        def _(i):
            c128 = pl.multiple_of(tbuf[i], 128)
            row0 = pl.multiple_of(base + (i // 8) * 8, 8)
            pltpu.async_copy(
                out_hbm.at[pl.ds(row0, 8), pl.ds(c128, 128)],
                g_hbm.at[pl.ds(pl.multiple_of((base + i) * 8, 8), 8), :],
                sem_g,
            )

        pltpu.make_async_copy(
            out_hbm.at[pl.ds(0, 8 * per_core), pl.ds(0, 128)],
            g_hbm.at[pl.ds(8 * base, 8 * per_core), :],
            sem_g,
        ).wait()

    return gather_kernel(output, cols128)


def _combine_body(p_ref, s_ref, off_ref, g_ref, res_ref, *, b, smooth):
    off = off_ref[...]
    g = g_ref[...]
    lane = jax.lax.broadcasted_iota(jnp.int32, g.shape, 1)
    gsum = jnp.sum(jnp.where(lane == off, g, 0.0))
    const = b * (_xlogy_const(_CONFIDENCE) - _xlogy_const(smooth))
    ptot = jnp.sum(p_ref[...]) / 128.0
    scsum = jnp.sum(s_ref[...])
    res_ref[0, 0] = (ptot - smooth * scsum + const
                     - (_CONFIDENCE - smooth) * gsum)


@jax.jit
def kernel(output, target, one_hot):
    b, v = output.shape
    smooth = _LABEL_SMOOTHING / (v - 2)
    tgt = target.astype(jnp.int32)
    cols128 = tgt & ~127
    i = jnp.arange(8 * b, dtype=jnp.int32)
    off = jnp.where(
        i % 8 == (i // 8) % 8, jnp.repeat(tgt & 127, 8), -1
    ).reshape(8 * b, 1)

    g = _sc_gather(output, cols128)
    ssum = _sc_rowsum(output)
    parts = _dense_partial(one_hot, output, _B_SC)
    parts = parts.reshape(parts.shape[0], 128)
    nb = parts.shape[0]

    res = pl.pallas_call(
        functools.partial(_combine_body, b=b, smooth=smooth),
        in_specs=[
            pl.BlockSpec((nb, 128), lambda: (0, 0)),
            pl.BlockSpec((_NW, 16), lambda: (0, 0)),
            pl.BlockSpec((8 * b, 1), lambda: (0, 0)),
            pl.BlockSpec((8 * b, 128), lambda: (0, 0)),
        ],
        out_specs=pl.BlockSpec(memory_space=pltpu.SMEM),
        out_shape=jax.ShapeDtypeStruct((1, 1), jnp.float32),
    )(parts, ssum, off, g)
    return res[0, 0]


# speedup vs baseline: 1.1056x; 1.1056x over previous
import functools

import jax
import jax.numpy as jnp
from jax import lax
from jax.experimental import pallas as pl
from jax.experimental.pallas import tpu as pltpu
from jax.experimental.pallas import tpu_sc as plsc

_LABEL_SMOOTHING = 0.1
_CONFIDENCE = 1.0 - _LABEL_SMOOTHING
_RB = 32
_NC = 2
_NSUB = 16
_NW = _NC * _NSUB
_B_SC = 768
_CHUNK = 2048


def _xlogy(x):
    safe = jnp.where(x > 0, x, 1.0)
    return jnp.where(x > 0, x * jnp.log(safe), 0.0)


def _xlogy_const(x):
    import math
    return x * math.log(x) if x > 0 else 0.0


def _dense_body(h_ref, out_ref, res_ref, *, b):
    j = pl.program_id(0)
    blk = out_ref[...]
    h = h_ref[...]
    colsum = jnp.sum(blk, axis=0, keepdims=True)
    res_ref[...] = jnp.full((1, 1, 128), -jnp.sum(colsum * h),
                            dtype=jnp.float32)

    @pl.when(j == 0)
    def _():
        res_ref[...] += jnp.full((1, 1, 128), b * jnp.sum(_xlogy(h)),
                                 dtype=jnp.float32)


def _dense_partial(one_hot, output, rows):
    b, v = output.shape
    nb = rows // _RB
    return pl.pallas_call(
        functools.partial(_dense_body, b=b),
        grid=(nb,),
        in_specs=[
            pl.BlockSpec((1, v), lambda j: (0, 0)),
            pl.BlockSpec((_RB, v), lambda j: (j, 0)),
        ],
        out_specs=pl.BlockSpec((1, 1, 128), lambda j: (j, 0, 0)),
        out_shape=jax.ShapeDtypeStruct((nb, 1, 128), jnp.float32),
        compiler_params=pltpu.CompilerParams(
            dimension_semantics=("arbitrary",),
        ),
    )(one_hot, output)


def _sc_rowsum(output):
    b, v = output.shape
    rows_pw = (b - _B_SC) // _NW
    trs_pw = rows_pw // 8
    n_full = (v // 128) // (_CHUNK // 128)
    v_pad = -(-v // 128) * 128
    tail_dma = v_pad - n_full * _CHUNK
    tail_valid = v - n_full * _CHUNK
    nch = trs_pw * n_full

    mesh = plsc.VectorSubcoreMesh(core_axis_name="c", subcore_axis_name="s")

    @functools.partial(
        pl.kernel,
        out_type=jax.ShapeDtypeStruct((_NW, 16), jnp.float32),
        mesh=mesh,
        scratch_types=[
            pltpu.VMEM((8, _CHUNK), jnp.float32),
            pltpu.VMEM((8, _CHUNK), jnp.float32),
            pltpu.VMEM((8, tail_dma), jnp.float32),
            pltpu.VMEM((16,), jnp.float32),
            pltpu.SemaphoreType.DMA,
            pltpu.SemaphoreType.DMA,
            pltpu.SemaphoreType.DMA,
        ],
    )
    def rowsum_kernel(out_hbm, o_hbm, buf0, buf1, tbuf, acc_ref,
                      sem0, sem1, sem2):
        w = lax.axis_index("s") * _NC + lax.axis_index("c")
        row_base = _B_SC + w * rows_pw
        acc_ref[...] = jnp.zeros((16,), jnp.float32)

        def chunk_src(g):
            row = pl.multiple_of(row_base + (g // n_full) * 8, 8)
            col = pl.multiple_of((g % n_full) * _CHUNK, 128)
            return out_hbm.at[pl.ds(row, 8), pl.ds(col, _CHUNK)]

        def accum(bref, ncols):
            @pl.loop(0, ncols // 32)
            def _(cc):
                col = cc * 32
                s = None
                for dc in (0, 16):
                    vals = [bref[r, pl.ds(col + dc, 16)] for r in range(8)]
                    t = ((vals[0] + vals[1]) + (vals[2] + vals[3])) + (
                        (vals[4] + vals[5]) + (vals[6] + vals[7]))
                    s = t if s is None else s + t
                acc_ref[...] += s

        pltpu.async_copy(chunk_src(0), buf0, sem0)
        pltpu.async_copy(chunk_src(1), buf1, sem1)

        @pl.loop(0, nch // 2)
        def _(kk):
            g = kk * 2
            pltpu.make_async_copy(chunk_src(g), buf0, sem0).wait()
            accum(buf0, _CHUNK)

            @pl.when(g + 2 < nch)
            def _():
                pltpu.async_copy(chunk_src(g + 2), buf0, sem0)

            pltpu.make_async_copy(chunk_src(g + 1), buf1, sem1).wait()
            accum(buf1, _CHUNK)

            @pl.when(g + 3 < nch)
            def _():
                pltpu.async_copy(chunk_src(g + 3), buf1, sem1)

        for tr in range(trs_pw):
            row = pl.multiple_of(row_base + tr * 8, 8)
            col = pl.multiple_of(n_full * _CHUNK, 128)
            pltpu.async_copy(
                out_hbm.at[pl.ds(row, 8), pl.ds(col, tail_dma)],
                tbuf, sem2,
            ).wait()
            accum(tbuf, tail_valid)

        pltpu.sync_copy(acc_ref, o_hbm.at[w])

    return rowsum_kernel(output)


def _sc_gather(output, cols128):
    b, v = output.shape
    per_core = b // _NC

    mesh = plsc.ScalarSubcoreMesh(axis_name="c", num_cores=_NC)

    @functools.partial(
        pl.kernel,
        out_type=jax.ShapeDtypeStruct((8 * b, 128), jnp.float32),
        mesh=mesh,
        scratch_types=[
            pltpu.SMEM((per_core,), jnp.int32),
            pltpu.SemaphoreType.DMA,
            pltpu.SemaphoreType.DMA,
        ],
    )
    def gather_kernel(out_hbm, c128_hbm, g_hbm, tbuf, sem_t, sem_g):
        cid = lax.axis_index("c")
        base = cid * per_core
        pltpu.async_copy(
            c128_hbm.at[pl.ds(base, per_core)], tbuf, sem_t
        ).wait()

        @pl.loop(0, per_core)
        def _(i):
            c128 = pl.multiple_of(tbuf[i], 128)
            row0 = pl.multiple_of(base + (i // 8) * 8, 8)
            pltpu.async_copy(
                out_hbm.at[pl.ds(row0, 8), pl.ds(c128, 128)],
                g_hbm.at[pl.ds(pl.multiple_of((base + i) * 8, 8), 8), :],
                sem_g,
            )

        pltpu.make_async_copy(
            out_hbm.at[pl.ds(0, 8 * per_core), pl.ds(0, 128)],
            g_hbm.at[pl.ds(8 * base, 8 * per_core), :],
            sem_g,
        ).wait()

    return gather_kernel(output, cols128)


def _combine_body(p_ref, s_ref, off_ref, g_ref, res_ref, *, b, smooth):
    off = off_ref[...]
    g = g_ref[...]
    lane = jax.lax.broadcasted_iota(jnp.int32, g.shape, 1)
    gsum = jnp.sum(jnp.where(lane == off, g, 0.0))
    const = b * (_xlogy_const(_CONFIDENCE) - _xlogy_const(smooth))
    ptot = jnp.sum(p_ref[...]) / 128.0
    scsum = jnp.sum(s_ref[...])
    res_ref[0, 0] = (ptot - smooth * scsum + const
                     - (_CONFIDENCE - smooth) * gsum)


@jax.jit
def kernel(output, target, one_hot):
    b, v = output.shape
    smooth = _LABEL_SMOOTHING / (v - 2)
    tgt = target.astype(jnp.int32)
    cols128 = tgt & ~127
    i = jnp.arange(8 * b, dtype=jnp.int32)
    off = jnp.where(
        i % 8 == (i // 8) % 8, jnp.repeat(tgt & 127, 8), -1
    ).reshape(8 * b, 1)

    g = _sc_gather(output, cols128)
    ssum = _sc_rowsum(output)
    parts = _dense_partial(one_hot, output, _B_SC)
    parts = parts.reshape(parts.shape[0], 128)
    nb = parts.shape[0]

    res = pl.pallas_call(
        functools.partial(_combine_body, b=b, smooth=smooth),
        in_specs=[
            pl.BlockSpec((nb, 128), lambda: (0, 0)),
            pl.BlockSpec((_NW, 16), lambda: (0, 0)),
            pl.BlockSpec((8 * b, 1), lambda: (0, 0)),
            pl.BlockSpec((8 * b, 128), lambda: (0, 0)),
        ],
        out_specs=pl.BlockSpec(memory_space=pltpu.SMEM),
        out_shape=jax.ShapeDtypeStruct((1, 1), jnp.float32),
    )(parts, ssum, off, g)
    return res[0, 0]


# speedup vs baseline: 1.1651x; 1.0538x over previous
import functools

import jax
import jax.numpy as jnp
from jax import lax
from jax.experimental import pallas as pl
from jax.experimental.pallas import tpu as pltpu
from jax.experimental.pallas import tpu_sc as plsc

_LABEL_SMOOTHING = 0.1
_CONFIDENCE = 1.0 - _LABEL_SMOOTHING
_RB = 32
_NC = 2
_NSUB = 16
_NW = _NC * _NSUB
_B_TC = 512
_CHUNK = 2048


def _xlogy(x):
    safe = jnp.where(x > 0, x, 1.0)
    return jnp.where(x > 0, x * jnp.log(safe), 0.0)


def _xlogy_const(x):
    import math
    return x * math.log(x) if x > 0 else 0.0


def _dense_body(h_ref, t_ref, out_ref, res_ref, *, b, v, smooth):
    j = pl.program_id(0)
    blk = out_ref[...]
    h = h_ref[...]
    tgt = t_ref[...]
    colsum = jnp.sum(blk, axis=0, keepdims=True)
    cols = jax.lax.broadcasted_iota(jnp.int32, (1, v), 1)
    hit = cols == tgt
    gpart = jnp.sum(jnp.where(hit, blk, 0.0))
    val = -jnp.sum(colsum * h) - (_CONFIDENCE - smooth) * gpart
    res_ref[...] = jnp.full((1, 1, 128), val, dtype=jnp.float32)

    @pl.when(j == 0)
    def _():
        res_ref[...] += jnp.full((1, 1, 128), b * jnp.sum(_xlogy(h)),
                                 dtype=jnp.float32)


def _dense_partial(one_hot, target2d, output, rows):
    b, v = output.shape
    smooth = _LABEL_SMOOTHING / (v - 2)
    nb = rows // _RB
    return pl.pallas_call(
        functools.partial(_dense_body, b=b, v=v, smooth=smooth),
        grid=(nb,),
        in_specs=[
            pl.BlockSpec((1, v), lambda j: (0, 0)),
            pl.BlockSpec((_RB, 1), lambda j: (j, 0)),
            pl.BlockSpec((_RB, v), lambda j: (j, 0)),
        ],
        out_specs=pl.BlockSpec((1, 1, 128), lambda j: (j, 0, 0)),
        out_shape=jax.ShapeDtypeStruct((nb, 1, 128), jnp.float32),
        compiler_params=pltpu.CompilerParams(
            dimension_semantics=("arbitrary",),
        ),
    )(one_hot, target2d, output)


def _sc_rowsum_gather(output, tsplat):
    b, v = output.shape
    rows_pw = (b - _B_TC) // _NW
    trs_pw = rows_pw // 8
    n_full = (v // 128) // (_CHUNK // 128)
    v_pad = -(-v // 128) * 128
    tail_dma = v_pad - n_full * _CHUNK
    tail_valid = v - n_full * _CHUNK
    nch = trs_pw * n_full

    mesh = plsc.VectorSubcoreMesh(core_axis_name="c", subcore_axis_name="s")

    @functools.partial(
        pl.kernel,
        out_type=jax.ShapeDtypeStruct((_NW, 32), jnp.float32),
        mesh=mesh,
        scratch_types=[
            pltpu.VMEM((8, _CHUNK), jnp.float32),
            pltpu.VMEM((8, _CHUNK), jnp.float32),
            pltpu.VMEM((8, tail_dma), jnp.float32),
            pltpu.VMEM((rows_pw, 128), jnp.int32),
            pltpu.VMEM((32,), jnp.float32),
            pltpu.SemaphoreType.DMA,
            pltpu.SemaphoreType.DMA,
            pltpu.SemaphoreType.DMA,
        ],
    )
    def rowsum_kernel(out_hbm, t_hbm, o_hbm, buf0, buf1, tbuf, tv, acc_ref,
                      sem0, sem1, sem2):
        w = lax.axis_index("s") * _NC + lax.axis_index("c")
        row_base = _B_TC + w * rows_pw
        acc_ref[...] = jnp.zeros((32,), jnp.float32)
        pltpu.async_copy(
            t_hbm.at[pl.ds(row_base, rows_pw)], tv, sem2
        ).wait()
        ivec = lax.iota(jnp.int32, 16)

        def chunk_src(g):
            row = pl.multiple_of(row_base + (g // n_full) * 8, 8)
            col = pl.multiple_of((g % n_full) * _CHUNK, 128)
            return out_hbm.at[pl.ds(row, 8), pl.ds(col, _CHUNK)]

        def accum(bref, ncols, col0, r0):
            @pl.loop(0, ncols // 32)
            def _(cc):
                col = cc * 32
                s = None
                gs = None
                for dc in (0, 16):
                    cg = col0 + col + dc + ivec
                    vals = [bref[r, pl.ds(col + dc, 16)] for r in range(8)]
                    t = ((vals[0] + vals[1]) + (vals[2] + vals[3])) + (
                        (vals[4] + vals[5]) + (vals[6] + vals[7]))
                    s = t if s is None else s + t
                    hits = [
                        jnp.where(cg == tv[r0 + r, 0:16], vals[r], 0.0)
                        for r in range(8)
                    ]
                    gt = ((hits[0] + hits[1]) + (hits[2] + hits[3])) + (
                        (hits[4] + hits[5]) + (hits[6] + hits[7]))
                    gs = gt if gs is None else gs + gt
                acc_ref[0:16] += s
                acc_ref[16:32] += gs

        pltpu.async_copy(chunk_src(0), buf0, sem0)
        pltpu.async_copy(chunk_src(1), buf1, sem1)

        @pl.loop(0, nch // 2)
        def _(kk):
            g = kk * 2
            pltpu.make_async_copy(chunk_src(g), buf0, sem0).wait()
            accum(buf0, _CHUNK, (g % n_full) * _CHUNK, (g // n_full) * 8)

            @pl.when(g + 2 < nch)
            def _():
                pltpu.async_copy(chunk_src(g + 2), buf0, sem0)

            g1 = g + 1
            pltpu.make_async_copy(chunk_src(g1), buf1, sem1).wait()
            accum(buf1, _CHUNK, (g1 % n_full) * _CHUNK, (g1 // n_full) * 8)

            @pl.when(g + 3 < nch)
            def _():
                pltpu.async_copy(chunk_src(g + 3), buf1, sem1)

        for tr in range(trs_pw):
            row = pl.multiple_of(row_base + tr * 8, 8)
            col = pl.multiple_of(n_full * _CHUNK, 128)
            pltpu.async_copy(
                out_hbm.at[pl.ds(row, 8), pl.ds(col, tail_dma)],
                tbuf, sem2,
            ).wait()
            accum(tbuf, tail_valid, n_full * _CHUNK, tr * 8)

        pltpu.sync_copy(acc_ref, o_hbm.at[w])

    return rowsum_kernel(output, tsplat)


def _combine_body(p_ref, s_ref, res_ref, *, b, smooth):
    const = b * (_xlogy_const(_CONFIDENCE) - _xlogy_const(smooth))
    ptot = jnp.sum(p_ref[...]) / 128.0
    sc = s_ref[...]
    scsum = jnp.sum(sc[:, 0:16])
    gsum = jnp.sum(sc[:, 16:32])
    res_ref[0, 0] = (ptot + const - smooth * scsum
                     - (_CONFIDENCE - smooth) * gsum)


@jax.jit
def kernel(output, target, one_hot):
    b, v = output.shape
    smooth = _LABEL_SMOOTHING / (v - 2)
    tgt = target.astype(jnp.int32)
    tsplat = jnp.broadcast_to(tgt[:, None], (b, 128))

    ssum = _sc_rowsum_gather(output, tsplat)
    parts = _dense_partial(one_hot, tgt.reshape(b, 1), output, _B_TC)
    parts = parts.reshape(parts.shape[0], 128)
    nb = parts.shape[0]

    res = pl.pallas_call(
        functools.partial(_combine_body, b=b, smooth=smooth),
        in_specs=[
            pl.BlockSpec((nb, 128), lambda: (0, 0)),
            pl.BlockSpec((_NW, 32), lambda: (0, 0)),
        ],
        out_specs=pl.BlockSpec(memory_space=pltpu.SMEM),
        out_shape=jax.ShapeDtypeStruct((1, 1), jnp.float32),
    )(parts, ssum)
    return res[0, 0]


# speedup vs baseline: 1.2447x; 1.0683x over previous
import functools

import jax
import jax.numpy as jnp
from jax import lax
from jax.experimental import pallas as pl
from jax.experimental.pallas import tpu as pltpu
from jax.experimental.pallas import tpu_sc as plsc

_LABEL_SMOOTHING = 0.1
_CONFIDENCE = 1.0 - _LABEL_SMOOTHING
_RB = 32
_NS = 1
_NC = 2


def _xlogy(x):
    safe = jnp.where(x > 0, x, 1.0)
    return jnp.where(x > 0, x * jnp.log(safe), 0.0)


def _xlogy_const(x):
    import math
    return x * math.log(x) if x > 0 else 0.0


def _dense_body(h_ref, *refs, b):
    j = pl.program_id(0)
    blk_refs, res_ref = refs[:-1], refs[-1]
    h = h_ref[...]

    colsum = jnp.sum(blk_refs[0][...], axis=0, keepdims=True)
    for r in blk_refs[1:]:
        colsum += jnp.sum(r[...], axis=0, keepdims=True)
    res_ref[...] = jnp.full((1, 1, 128), -jnp.sum(colsum * h),
                            dtype=jnp.float32)

    @pl.when(j == 0)
    def _():
        res_ref[...] += jnp.full((1, 1, 128), b * jnp.sum(_xlogy(h)),
                                 dtype=jnp.float32)


def _dense_partial(one_hot, output):
    b, v = output.shape
    nb = b // (_RB * _NS)
    in_specs = [pl.BlockSpec((1, v), lambda j: (0, 0))]
    for q in range(_NS):
        in_specs.append(
            pl.BlockSpec((_RB, v), functools.partial(
                lambda j, q: (_NS * j + q, 0), q=q))
        )
    return pl.pallas_call(
        functools.partial(_dense_body, b=b),
        grid=(nb,),
        in_specs=in_specs,
        out_specs=pl.BlockSpec((1, 1, 128), lambda j: (j, 0, 0)),
        out_shape=jax.ShapeDtypeStruct((nb, 1, 128), jnp.float32),
        compiler_params=pltpu.CompilerParams(
            dimension_semantics=("parallel",),
        ),
    )(one_hot, *([output] * _NS))


def _sc_gather(output, cols128):
    b, v = output.shape
    per_core = b // _NC

    mesh = plsc.ScalarSubcoreMesh(axis_name="c", num_cores=_NC)

    @functools.partial(
        pl.kernel,
        out_type=jax.ShapeDtypeStruct((8 * b, 128), jnp.float32),
        mesh=mesh,
        scratch_types=[
            pltpu.SMEM((per_core,), jnp.int32),
            pltpu.SemaphoreType.DMA,
            pltpu.SemaphoreType.DMA,
        ],
    )
    def gather_kernel(out_hbm, c128_hbm, g_hbm, tbuf, sem_t, sem_g):
        cid = lax.axis_index("c")
        base = cid * per_core
        pltpu.async_copy(
            c128_hbm.at[pl.ds(base, per_core)], tbuf, sem_t
        ).wait()

        @pl.loop(0, per_core)
        def _(i):
            c128 = pl.multiple_of(tbuf[i], 128)
            row0 = pl.multiple_of(base + (i // 8) * 8, 8)
            pltpu.async_copy(
                out_hbm.at[pl.ds(row0, 8), pl.ds(c128, 128)],
                g_hbm.at[pl.ds(pl.multiple_of((base + i) * 8, 8), 8), :],
                sem_g,
            )

        pltpu.make_async_copy(
            out_hbm.at[pl.ds(0, 8 * per_core), pl.ds(0, 128)],
            g_hbm.at[pl.ds(8 * base, 8 * per_core), :],
            sem_g,
        ).wait()

    return gather_kernel(output, cols128)


def _combine_body(p_ref, off_ref, g_ref, res_ref, *, b, smooth):
    off = off_ref[...]
    g = g_ref[...]
    lane = jax.lax.broadcasted_iota(jnp.int32, g.shape, 1)
    gsum = jnp.sum(jnp.where(lane == off, g, 0.0))
    const = b * (_xlogy_const(_CONFIDENCE) - _xlogy_const(smooth))
    ptot = jnp.sum(p_ref[...]) / 128.0
    res_ref[0, 0] = ptot + const - (_CONFIDENCE - smooth) * gsum


@jax.jit
def kernel(output, target, one_hot):
    b, v = output.shape
    smooth = _LABEL_SMOOTHING / (v - 2)
    tgt = target.astype(jnp.int32)
    cols128 = tgt & ~127
    i = jnp.arange(8 * b, dtype=jnp.int32)
    off = jnp.where(
        i % 8 == (i // 8) % 8, jnp.repeat(tgt & 127, 8), -1
    ).reshape(8 * b, 1)

    g = _sc_gather(output, cols128)
    parts = _dense_partial(one_hot, output)
    parts = parts.reshape(parts.shape[0], 128)
    nb = parts.shape[0]

    res = pl.pallas_call(
        functools.partial(_combine_body, b=b, smooth=smooth),
        in_specs=[
            pl.BlockSpec((nb, 128), lambda: (0, 0)),
            pl.BlockSpec((8 * b, 1), lambda: (0, 0)),
            pl.BlockSpec((8 * b, 128), lambda: (0, 0)),
        ],
        out_specs=pl.BlockSpec(memory_space=pltpu.SMEM),
        out_shape=jax.ShapeDtypeStruct((1, 1), jnp.float32),
    )(parts, off, g)
    return res[0, 0]
